# Initial kernel scaffold; baseline (speedup 1.0000x reference)
import jax
import jax.numpy as jnp
from jax.experimental import pallas as pl


def kernel(xyz, data):
    raise NotImplementedError("write your pallas kernel here")


# speedup vs baseline: 14.2707x; 14.2707x over previous
import functools

import jax
import jax.numpy as jnp
from jax import lax
from jax.experimental import pallas as pl
from jax.experimental.pallas import tpu as pltpu
from jax.experimental.pallas import tpu_sc as plsc

BS = 8
N = 2048
D = 128
K = 16
ROWS = 256
IMAX = 0x7FFFFFFF

SC_CORES = 2
SC_SUBCORES = 16
NW = SC_CORES * SC_SUBCORES
PTS_PER_W = (BS * N) // NW
CHUNK = 8


def _topk_body(lhs_ref, rhs_ref, sn_ref, sm_ref, out_ref):
    b = pl.program_id(0)
    p = jnp.dot(lhs_ref[0], rhs_ref[0], preferred_element_type=jnp.float32)
    d = -2.0 * p
    d = d + sn_ref[0]
    d = d + sm_ref[0]
    bits = lax.bitcast_convert_type(d, jnp.int32)
    iota = lax.broadcasted_iota(jnp.int32, (ROWS, N), 1)
    keys = (bits & (-2048)) | iota
    keys = jnp.where(d < 1e-8, IMAX, keys)
    cols = []
    for _ in range(K):
        m = jnp.min(keys, axis=1, keepdims=True)
        cols.append(m & 2047)
        keys = jnp.where(keys == m, IMAX, keys)
    out_ref[0] = jnp.concatenate(cols, axis=1) + b * N


def _tc_topk(lhs, rhs, sn, sm):
    grid = (BS, N // ROWS)
    return pl.pallas_call(
        _topk_body,
        grid=grid,
        in_specs=[
            pl.BlockSpec((1, ROWS, 8), lambda b, r: (b, r, 0)),
            pl.BlockSpec((1, 8, N), lambda b, r: (b, 0, 0)),
            pl.BlockSpec((1, ROWS, 1), lambda b, r: (b, r, 0)),
            pl.BlockSpec((1, 1, N), lambda b, r: (b, 0, 0)),
        ],
        out_specs=pl.BlockSpec((1, ROWS, K), lambda b, r: (b, r, 0)),
        out_shape=jax.ShapeDtypeStruct((BS, N, K), jnp.int32),
    )(lhs, rhs, sn, sm)


def _sc_gather_max(data2d, idx_flat):
    mesh = plsc.VectorSubcoreMesh(
        core_axis_name="c", subcore_axis_name="s",
        num_cores=SC_CORES, num_subcores=SC_SUBCORES)

    @functools.partial(
        pl.kernel,
        mesh=mesh,
        out_type=jax.ShapeDtypeStruct((BS * N, D), jnp.float32),
        scratch_types=[
            pltpu.VMEM((CHUNK * K,), jnp.int32),
            pltpu.VMEM((CHUNK * K, D), jnp.float32),
            pltpu.VMEM((CHUNK, D), jnp.float32),
            pltpu.VMEM((CHUNK, D), jnp.float32),
            pltpu.SemaphoreType.DMA,
        ],
    )
    def k(data_hbm, idx_hbm, out_hbm, idx_v, nbr_v, self_v, out_v, sem):
        wid = lax.axis_index("s") * SC_CORES + lax.axis_index("c")
        base = wid * PTS_PER_W

        def chunk_body(c, _):
            pb = base + c * CHUNK
            pltpu.sync_copy(idx_hbm.at[pl.ds(pb * K, CHUNK * K)], idx_v)
            pltpu.async_copy(data_hbm.at[idx_v], nbr_v, sem).wait()
            pltpu.sync_copy(data_hbm.at[pl.ds(pb, CHUNK)], self_v)
            for p in range(CHUNK):
                for f in range(D // 16):
                    acc = self_v[p, pl.ds(f * 16, 16)]
                    for j in range(K):
                        acc = jnp.maximum(
                            acc, nbr_v[p * K + j, pl.ds(f * 16, 16)])
                    out_v[p, pl.ds(f * 16, 16)] = acc
            pltpu.sync_copy(out_v, out_hbm.at[pl.ds(pb, CHUNK)])
            return ()

        lax.fori_loop(0, PTS_PER_W // CHUNK, chunk_body, (), unroll=False)

    return k(data2d, idx_flat)


def kernel(xyz, data):
    xb = xyz.astype(jnp.bfloat16)
    lhs = jnp.pad(xb, ((0, 0), (0, 0), (0, 5)))
    rhs = jnp.transpose(lhs, (0, 2, 1))
    s = jnp.sum(xyz ** 2, axis=-1)
    idx = _tc_topk(lhs, rhs, s[:, :, None], s[:, None, :])
    pooled = _sc_gather_max(data.reshape(BS * N, D), idx.reshape(-1))
    return (xyz, pooled.reshape(BS, N, D))
